# baseline (speedup 1.0000x reference)
import jax
import jax.numpy as jnp
from jax import lax
from jax.experimental import pallas as pl
from jax.experimental.pallas import tpu as pltpu

_GLOB = 64


def _dot(a, b, dims):
    return lax.dot_general(a, b, dimension_numbers=(dims, ((), ())),
                           preferred_element_type=jnp.float32)


_BB = 8


def _fused_kernel(src_ref, tgt_ref, ew1_ref, eb1_ref, ew2_ref, eb2_ref,
                  pw0_ref, pb0_ref, pw1_ref, pb1_ref,
                  w0f_ref, w0gs_ref, w0go_ref, olb0_ref, w1t_ref, olb1_ref,
                  t0_ref, xol_ref, yol_ref, xsc_ref, ysc_ref,
                  psrc_ref, pnrm_ref):
    ew1 = ew1_ref[...]
    eb1 = eb1_ref[...]
    ew2 = ew2_ref[...]
    eb2 = eb2_ref[...]
    w0f = w0f_ref[...]
    w1t = w1t_ref[...]
    b1c = olb1_ref[...]

    def encode(pts):
        h = _dot(ew1, pts[0:3], ((1,), (0,))) + eb1
        h = jnp.maximum(h, 0.0)
        f = _dot(ew2, h.astype(jnp.bfloat16), ((1,), (0,))) + eb2
        return jnp.maximum(f, 0.0)

    def overlap(i, f, g_self, g_other, ol_out, sc_out):
        gbias = (_dot(w0gs_ref[...], g_self, ((1,), (0,)))
                 + _dot(w0go_ref[...], g_other, ((1,), (0,)))
                 + olb0_ref[...])
        h = _dot(w0f, f.astype(jnp.bfloat16), ((1,), (0,)))
        h = jnp.maximum(h + gbias, 0.0)
        lt = _dot(w1t, h.astype(jnp.bfloat16), ((1,), (0,))) + b1c
        ol_out[i] = lt
        sc_out[i] = 1.0 / (1.0 + jnp.exp(lt[0:1, :] - lt[1:2, :]))

    fs, ft, sg, tg = [], [], [], []
    for i in range(_BB):
        fs.append(encode(src_ref[i]))
        ft.append(encode(tgt_ref[i]))
        sg.append(jnp.max(fs[i], axis=1, keepdims=True))
        tg.append(jnp.max(ft[i], axis=1, keepdims=True))
        if i > 0:
            overlap(i - 1, fs[i - 1], sg[i - 1], tg[i - 1], xol_ref, xsc_ref)
            overlap(i - 1, ft[i - 1], tg[i - 1], sg[i - 1], yol_ref, ysc_ref)
    last = _BB - 1
    overlap(last, fs[last], sg[last], tg[last], xol_ref, xsc_ref)
    overlap(last, ft[last], tg[last], sg[last], yol_ref, ysc_ref)

    for i in range(_BB):
        hcat = jnp.concatenate([sg[i], tg[i]], axis=0)
        ph = _dot(pw0_ref[...], hcat, ((1,), (0,))) + pb0_ref[...]
        ph = jnp.maximum(ph, 0.0)
        pose = _dot(pw1_ref[...], ph, ((1,), (0,))) + pb1_ref[...]

        q0 = pose[0:1, 0:1] + 1.0
        qx = pose[1:2, 0:1]
        qy = pose[2:3, 0:1]
        qz = pose[3:4, 0:1]
        inv = 1.0 / (jnp.sqrt(q0 * q0 + qx * qx + qy * qy + qz * qz) + 1e-8)
        w = q0 * inv
        x = qx * inv
        y = qy * inv
        z = qz * inv
        r00 = 1.0 - 2.0 * (y * y + z * z)
        r01 = 2.0 * (x * y - w * z)
        r02 = 2.0 * (x * z + w * y)
        r10 = 2.0 * (x * y + w * z)
        r11 = 1.0 - 2.0 * (x * x + z * z)
        r12 = 2.0 * (y * z - w * x)
        r20 = 2.0 * (x * z - w * y)
        r21 = 2.0 * (y * z + w * x)
        r22 = 1.0 - 2.0 * (x * x + y * y)
        tx = pose[4:5, 0:1]
        ty = pose[5:6, 0:1]
        tz = pose[6:7, 0:1]

        z3 = jnp.zeros((1, 3), jnp.float32)
        row0 = jnp.concatenate([r00, r01, r02], axis=1)
        row1 = jnp.concatenate([r10, r11, r12], axis=1)
        row2 = jnp.concatenate([r20, r21, r22], axis=1)
        w6 = jnp.concatenate([
            jnp.concatenate([row0, z3], axis=1),
            jnp.concatenate([row1, z3], axis=1),
            jnp.concatenate([row2, z3], axis=1),
            jnp.concatenate([z3, row0], axis=1),
            jnp.concatenate([z3, row1], axis=1),
            jnp.concatenate([z3, row2], axis=1),
        ], axis=0)
        tcol = jnp.concatenate([tx, ty, tz], axis=0)
        out6 = _dot(w6, src_ref[i], ((1,), (0,)))
        psrc_ref[i] = out6[0:3] + tcol
        pnrm_ref[i] = out6[3:6]

        one = jnp.ones((1, 1), jnp.float32)
        t0_ref[i] = jnp.concatenate([
            jnp.concatenate([r00, r01, r02, tx], axis=1),
            jnp.concatenate([r10, r11, r12, ty], axis=1),
            jnp.concatenate([r20, r21, r22, tz], axis=1),
            jnp.concatenate([z3, one], axis=1),
        ], axis=0)


@jax.jit
def _forward(enc_w0, enc_b0, enc_w1, enc_b1,
             pose_w0, pose_b0, pose_w1, pose_b1,
             ol_w0, ol_b0, ol_w1, ol_b1, src, tgt):
    B, N, _ = src.shape
    M = tgt.shape[1]
    f32 = jnp.float32

    ew1 = enc_w0.T.astype(f32)
    eb1 = enc_b0.reshape(-1, 1).astype(f32)
    ew2 = enc_w1.T.astype(jnp.bfloat16)
    eb2 = enc_b1.reshape(-1, 1).astype(f32)
    pw0 = pose_w0.T.astype(f32)
    pb0 = pose_b0.reshape(-1, 1).astype(f32)
    pw1 = pose_w1.T.astype(f32)
    pb1 = pose_b1.reshape(-1, 1).astype(f32)
    w0f = ol_w0[:_GLOB].T.astype(jnp.bfloat16)
    w0gs = ol_w0[_GLOB:2 * _GLOB].T.astype(f32)
    w0go = ol_w0[2 * _GLOB:].T.astype(f32)
    olb0 = ol_b0.reshape(-1, 1).astype(f32)
    w1t = jnp.transpose(ol_w1).astype(jnp.bfloat16)
    olb1 = ol_b1.reshape(2, 1).astype(f32)

    whole = lambda shape: pl.BlockSpec(shape, lambda b: (0,) * len(shape))

    src_T = jnp.transpose(src.astype(f32), (0, 2, 1))
    tgt_T = jnp.transpose(tgt[..., :3].astype(f32), (0, 2, 1))

    t0, x_ol, y_ol, x_sc, y_sc, src_tT, nrm_tT = pl.pallas_call(
        _fused_kernel,
        out_shape=(
            jax.ShapeDtypeStruct((B, 4, 4), f32),
            jax.ShapeDtypeStruct((B, 2, N), f32),
            jax.ShapeDtypeStruct((B, 2, M), f32),
            jax.ShapeDtypeStruct((B, 1, N), f32),
            jax.ShapeDtypeStruct((B, 1, M), f32),
            jax.ShapeDtypeStruct((B, 3, N), f32),
            jax.ShapeDtypeStruct((B, 3, N), f32),
        ),
        grid=(B // _BB,),
        in_specs=[
            pl.BlockSpec((_BB, 6, N), lambda b: (b, 0, 0)),
            pl.BlockSpec((_BB, 3, M), lambda b: (b, 0, 0)),
            whole((32, 3)), whole((32, 1)), whole((64, 32)), whole((64, 1)),
            whole((64, 128)), whole((64, 1)), whole((7, 64)), whole((7, 1)),
            whole((64, 64)), whole((64, 64)), whole((64, 64)), whole((64, 1)),
            whole((2, 64)), whole((2, 1)),
        ],
        out_specs=(
            pl.BlockSpec((_BB, 4, 4), lambda b: (b, 0, 0)),
            pl.BlockSpec((_BB, 2, N), lambda b: (b, 0, 0)),
            pl.BlockSpec((_BB, 2, M), lambda b: (b, 0, 0)),
            pl.BlockSpec((_BB, 1, N), lambda b: (b, 0, 0)),
            pl.BlockSpec((_BB, 1, M), lambda b: (b, 0, 0)),
            pl.BlockSpec((_BB, 3, N), lambda b: (b, 0, 0)),
            pl.BlockSpec((_BB, 3, N), lambda b: (b, 0, 0)),
        ),
        compiler_params=pltpu.CompilerParams(
            dimension_semantics=("arbitrary",),
            allow_input_fusion=[True, True] + [False] * 14),
    )(src_T, tgt_T,
      ew1, eb1, ew2, eb2, pw0, pb0, pw1, pb1,
      w0f, w0gs, w0go, olb0, w1t, olb1)

    return {
        "pred_Ts": [t0],
        "pred_src": [jnp.transpose(src_tT, (0, 2, 1))],
        "x_ol": x_ol,
        "y_ol": y_ol,
        "x_ol_score": x_sc[:, 0, :],
        "y_ol_score": y_sc[:, 0, :],
        "normal_src_t": jnp.transpose(nrm_tT, (0, 2, 1)),
        "normal_tgt": tgt[..., 3:],
    }


def kernel(enc_w0, enc_b0, enc_w1, enc_b1, pose_w0, pose_b0, pose_w1,
           pose_b1, ol_w0, ol_b0, ol_w1, ol_b1, src, tgt):
    return _forward(enc_w0, enc_b0, enc_w1, enc_b1,
                    pose_w0, pose_b0, pose_w1, pose_b1,
                    ol_w0, ol_b0, ol_w1, ol_b1, src, tgt)


# speedup vs baseline: 1.0090x; 1.0090x over previous
import jax
import jax.numpy as jnp
from jax import lax
from jax.experimental import pallas as pl
from jax.experimental.pallas import tpu as pltpu

_GLOB = 64


def _dot(a, b, dims):
    return lax.dot_general(a, b, dimension_numbers=(dims, ((), ())),
                           preferred_element_type=jnp.float32)


_BB = 8


def _fused_kernel(src_ref, tgt_ref, ew1_ref, eb1_ref, ew2_ref, eb2_ref,
                  pw0_ref, pb0_ref, pw1_ref, pb1_ref,
                  w0f_ref, w0gs_ref, w0go_ref, olb0_ref, w1t_ref, olb1_ref,
                  t0_ref, xol_ref, yol_ref, xsc_ref, ysc_ref,
                  psrc_ref, pnrm_ref):
    ew1 = ew1_ref[...]
    eb1 = eb1_ref[...]
    ew2 = ew2_ref[...]
    eb2 = eb2_ref[...]
    w0f = w0f_ref[...]
    w1t = w1t_ref[...]
    b1c = olb1_ref[...]

    def encode(pts):
        h = _dot(ew1, pts[0:3], ((1,), (0,))) + eb1
        h = jnp.maximum(h, 0.0)
        f = _dot(ew2, h.astype(jnp.bfloat16), ((1,), (0,))) + eb2
        return jnp.maximum(f, 0.0)

    fs, ft, sg, tg = [], [], [], []
    for i in range(_BB):
        fs.append(encode(src_ref[i]))
        ft.append(encode(tgt_ref[i]))
        sg.append(jnp.max(fs[i], axis=1, keepdims=True))
        tg.append(jnp.max(ft[i], axis=1, keepdims=True))

    def overlap(i, f, g_self, g_other, ol_out, sc_out):
        gbias = (_dot(w0gs_ref[...], g_self, ((1,), (0,)))
                 + _dot(w0go_ref[...], g_other, ((1,), (0,)))
                 + olb0_ref[...])
        h = _dot(w0f, f.astype(jnp.bfloat16), ((1,), (0,)))
        h = jnp.maximum(h + gbias, 0.0)
        lt = _dot(w1t, h.astype(jnp.bfloat16), ((1,), (0,))) + b1c
        ol_out[i] = lt
        sc_out[i] = 1.0 / (1.0 + jnp.exp(lt[0:1, :] - lt[1:2, :]))

    for i in range(_BB):
        overlap(i, fs[i], sg[i], tg[i], xol_ref, xsc_ref)
        overlap(i, ft[i], tg[i], sg[i], yol_ref, ysc_ref)

    for i in range(_BB):
        hcat = jnp.concatenate([sg[i], tg[i]], axis=0)
        ph = _dot(pw0_ref[...], hcat, ((1,), (0,))) + pb0_ref[...]
        ph = jnp.maximum(ph, 0.0)
        pose = _dot(pw1_ref[...], ph, ((1,), (0,))) + pb1_ref[...]

        q0 = pose[0:1, 0:1] + 1.0
        qx = pose[1:2, 0:1]
        qy = pose[2:3, 0:1]
        qz = pose[3:4, 0:1]
        inv = 1.0 / (jnp.sqrt(q0 * q0 + qx * qx + qy * qy + qz * qz) + 1e-8)
        w = q0 * inv
        x = qx * inv
        y = qy * inv
        z = qz * inv
        r00 = 1.0 - 2.0 * (y * y + z * z)
        r01 = 2.0 * (x * y - w * z)
        r02 = 2.0 * (x * z + w * y)
        r10 = 2.0 * (x * y + w * z)
        r11 = 1.0 - 2.0 * (x * x + z * z)
        r12 = 2.0 * (y * z - w * x)
        r20 = 2.0 * (x * z - w * y)
        r21 = 2.0 * (y * z + w * x)
        r22 = 1.0 - 2.0 * (x * x + y * y)
        tx = pose[4:5, 0:1]
        ty = pose[5:6, 0:1]
        tz = pose[6:7, 0:1]

        z3 = jnp.zeros((1, 3), jnp.float32)
        row0 = jnp.concatenate([r00, r01, r02], axis=1)
        row1 = jnp.concatenate([r10, r11, r12], axis=1)
        row2 = jnp.concatenate([r20, r21, r22], axis=1)
        w6 = jnp.concatenate([
            jnp.concatenate([row0, z3], axis=1),
            jnp.concatenate([row1, z3], axis=1),
            jnp.concatenate([row2, z3], axis=1),
            jnp.concatenate([z3, row0], axis=1),
            jnp.concatenate([z3, row1], axis=1),
            jnp.concatenate([z3, row2], axis=1),
        ], axis=0)
        tcol = jnp.concatenate([tx, ty, tz], axis=0)
        out6 = _dot(w6, src_ref[i], ((1,), (0,)))
        psrc_ref[i] = out6[0:3] + tcol
        pnrm_ref[i] = out6[3:6]

        one = jnp.ones((1, 1), jnp.float32)
        t0_ref[i] = jnp.concatenate([
            jnp.concatenate([r00, r01, r02, tx], axis=1),
            jnp.concatenate([r10, r11, r12, ty], axis=1),
            jnp.concatenate([r20, r21, r22, tz], axis=1),
            jnp.concatenate([z3, one], axis=1),
        ], axis=0)


@jax.jit
def _forward(enc_w0, enc_b0, enc_w1, enc_b1,
             pose_w0, pose_b0, pose_w1, pose_b1,
             ol_w0, ol_b0, ol_w1, ol_b1, src, tgt):
    B, N, _ = src.shape
    M = tgt.shape[1]
    f32 = jnp.float32

    ew1 = enc_w0.T.astype(f32)
    eb1 = enc_b0.reshape(-1, 1).astype(f32)
    ew2 = enc_w1.T.astype(jnp.bfloat16)
    eb2 = enc_b1.reshape(-1, 1).astype(f32)
    pw0 = pose_w0.T.astype(f32)
    pb0 = pose_b0.reshape(-1, 1).astype(f32)
    pw1 = pose_w1.T.astype(f32)
    pb1 = pose_b1.reshape(-1, 1).astype(f32)
    w0f = ol_w0[:_GLOB].T.astype(jnp.bfloat16)
    w0gs = ol_w0[_GLOB:2 * _GLOB].T.astype(f32)
    w0go = ol_w0[2 * _GLOB:].T.astype(f32)
    olb0 = ol_b0.reshape(-1, 1).astype(f32)
    w1t = jnp.transpose(ol_w1).astype(jnp.bfloat16)
    olb1 = ol_b1.reshape(2, 1).astype(f32)

    whole = lambda shape: pl.BlockSpec(shape, lambda b: (0,) * len(shape))

    src_T = jnp.transpose(src.astype(f32), (0, 2, 1))
    tgt_T = jnp.transpose(tgt[..., :3].astype(f32), (0, 2, 1))

    t0, x_ol, y_ol, x_sc, y_sc, src_tT, nrm_tT = pl.pallas_call(
        _fused_kernel,
        out_shape=(
            jax.ShapeDtypeStruct((B, 4, 4), f32),
            jax.ShapeDtypeStruct((B, 2, N), f32),
            jax.ShapeDtypeStruct((B, 2, M), f32),
            jax.ShapeDtypeStruct((B, 1, N), f32),
            jax.ShapeDtypeStruct((B, 1, M), f32),
            jax.ShapeDtypeStruct((B, 3, N), f32),
            jax.ShapeDtypeStruct((B, 3, N), f32),
        ),
        grid=(B // _BB,),
        in_specs=[
            pl.BlockSpec((_BB, 6, N), lambda b: (b, 0, 0)),
            pl.BlockSpec((_BB, 3, M), lambda b: (b, 0, 0)),
            whole((32, 3)), whole((32, 1)), whole((64, 32)), whole((64, 1)),
            whole((64, 128)), whole((64, 1)), whole((7, 64)), whole((7, 1)),
            whole((64, 64)), whole((64, 64)), whole((64, 64)), whole((64, 1)),
            whole((2, 64)), whole((2, 1)),
        ],
        out_specs=(
            pl.BlockSpec((_BB, 4, 4), lambda b: (b, 0, 0)),
            pl.BlockSpec((_BB, 2, N), lambda b: (b, 0, 0)),
            pl.BlockSpec((_BB, 2, M), lambda b: (b, 0, 0)),
            pl.BlockSpec((_BB, 1, N), lambda b: (b, 0, 0)),
            pl.BlockSpec((_BB, 1, M), lambda b: (b, 0, 0)),
            pl.BlockSpec((_BB, 3, N), lambda b: (b, 0, 0)),
            pl.BlockSpec((_BB, 3, N), lambda b: (b, 0, 0)),
        ),
        compiler_params=pltpu.CompilerParams(
            dimension_semantics=("arbitrary",),
            allow_input_fusion=[True, True] + [False] * 14),
    )(src_T, tgt_T,
      ew1, eb1, ew2, eb2, pw0, pb0, pw1, pb1,
      w0f, w0gs, w0go, olb0, w1t, olb1)

    return {
        "pred_Ts": [t0],
        "pred_src": [jnp.transpose(src_tT, (0, 2, 1))],
        "x_ol": x_ol,
        "y_ol": y_ol,
        "x_ol_score": x_sc[:, 0, :],
        "y_ol_score": y_sc[:, 0, :],
        "normal_src_t": jnp.transpose(nrm_tT, (0, 2, 1)),
        "normal_tgt": tgt[..., 3:],
    }


def kernel(enc_w0, enc_b0, enc_w1, enc_b1, pose_w0, pose_b0, pose_w1,
           pose_b1, ol_w0, ol_b0, ol_w1, ol_b1, src, tgt):
    return _forward(enc_w0, enc_b0, enc_w1, enc_b1,
                    pose_w0, pose_b0, pose_w1, pose_b1,
                    ol_w0, ol_b0, ol_w1, ol_b1, src, tgt)
